# Initial kernel scaffold; baseline (speedup 1.0000x reference)
import jax
import jax.numpy as jnp
from jax.experimental import pallas as pl


def kernel(x, edge_index, W1, b1, W2, b2, W3, b3):
    raise NotImplementedError("write your pallas kernel here")


# speedup vs baseline: 11.6800x; 11.6800x over previous
import functools

import jax
import jax.numpy as jnp
from jax import lax
from jax.experimental import pallas as pl
from jax.experimental.pallas import tpu as pltpu
from jax.experimental.pallas import tpu_sc as plsc

N = 10000
E = 160000
D = 256
NCLS = 40
NC, NS = 2, 16
K = 125
ROWS = E // K
RPT_FS = ROWS // NS
RPT_ES = ROWS // (NC * NS)
NWR = 10
WPT = N // NWR

_MESH = plsc.VectorSubcoreMesh(
    core_axis_name="c", subcore_axis_name="s", num_cores=NC, num_subcores=NS)

F32 = jnp.float32


@functools.partial(
    pl.kernel,
    out_type=(jax.ShapeDtypeStruct((N, 16), F32),) * 2,
    mesh=_MESH,
    compiler_params=pltpu.CompilerParams(use_tc_tiling_on_sc=False),
    scratch_types=[
        pltpu.VMEM((RPT_ES, K), jnp.int32),
        pltpu.VMEM((K, 16), F32),
        pltpu.VMEM((WPT, 16), F32),
        pltpu.VMEM_SHARED((N, 16), F32),
    ],
)
def _sc_deg(dst2, ones8, zeros8, d0_out, d1_out, idx_d, ones_v, zbuf, degsh):
    c = lax.axis_index("c")
    s = lax.axis_index("s")
    w = s * NC + c
    pltpu.sync_copy(dst2.at[pl.ds(w * RPT_ES, RPT_ES)], idx_d)
    pltpu.sync_copy(ones8, ones_v)

    @pl.when(s < NWR)
    def _():
        pltpu.sync_copy(zeros8, zbuf)
        pltpu.sync_copy(zbuf, degsh.at[pl.ds(s * WPT, WPT)])

    plsc.subcore_barrier()

    def it(j, carry):
        pltpu.sync_copy(ones_v, degsh.at[idx_d.at[j]], add=True)
        return carry

    lax.fori_loop(0, RPT_ES, it, 0)
    plsc.subcore_barrier()

    @pl.when((c == 0) & (s < NWR))
    def _():
        pltpu.sync_copy(degsh.at[pl.ds(s * WPT, WPT)],
                        d0_out.at[pl.ds(s * WPT, WPT)])

    @pl.when((c == 1) & (s < NWR))
    def _():
        pltpu.sync_copy(degsh.at[pl.ds(s * WPT, WPT)],
                        d1_out.at[pl.ds(s * WPT, WPT)])


@functools.partial(
    pl.kernel,
    out_type=(jax.ShapeDtypeStruct((N, 128), F32),) * 2,
    mesh=_MESH,
    compiler_params=pltpu.CompilerParams(use_tc_tiling_on_sc=False),
    scratch_types=[
        pltpu.VMEM((RPT_FS // 2, K), jnp.int32),
        pltpu.VMEM((RPT_FS // 2, K), jnp.int32),
        pltpu.VMEM((K, 128), F32),
        pltpu.VMEM((K, 128), F32),
        pltpu.VMEM_SHARED((N, 128), F32),
        pltpu.SemaphoreType.DMA,
        pltpu.SemaphoreType.DMA,
    ],
)
def _sc_apply_fs(y0, y1, src2, dst2, zeros128, z0_out, z1_out,
                 idx_s, idx_d, buf_a, buf_b, acc, sem_a, sem_b):
    c = lax.axis_index("c")
    s = lax.axis_index("s")
    half = RPT_FS // 2

    @pl.when(s < NWR)
    def _():
        pltpu.sync_copy(zeros128, buf_a)
        for k in range(WPT // K):
            pltpu.sync_copy(buf_a, acc.at[pl.ds(s * WPT + k * K, K)])

    plsc.subcore_barrier()

    def run(y):
        for p in range(2):
            pltpu.sync_copy(src2.at[pl.ds(s * RPT_FS + p * half, half)], idx_s)
            pltpu.sync_copy(dst2.at[pl.ds(s * RPT_FS + p * half, half)], idx_d)

            def it(j, carry):
                r = 2 * j
                d0 = pltpu.async_copy(y.at[idx_s.at[r]], buf_a, sem_a)
                d1 = pltpu.async_copy(y.at[idx_s.at[r + 1]], buf_b, sem_b)
                d0.wait()
                pltpu.sync_copy(buf_a, acc.at[idx_d.at[r]], add=True)
                d1.wait()
                pltpu.sync_copy(buf_b, acc.at[idx_d.at[r + 1]], add=True)
                return carry

            lax.fori_loop(0, half // 2, it, 0)

    @pl.when(c == 0)
    def _():
        run(y0)

    @pl.when(c == 1)
    def _():
        run(y1)

    plsc.subcore_barrier()

    @pl.when((c == 0) & (s < NWR))
    def _():
        pltpu.sync_copy(acc.at[pl.ds(s * WPT, WPT)],
                        z0_out.at[pl.ds(s * WPT, WPT)])

    @pl.when((c == 1) & (s < NWR))
    def _():
        pltpu.sync_copy(acc.at[pl.ds(s * WPT, WPT)],
                        z1_out.at[pl.ds(s * WPT, WPT)])


@functools.partial(
    pl.kernel,
    out_type=(jax.ShapeDtypeStruct((N, 64), F32),) * 2,
    mesh=_MESH,
    compiler_params=pltpu.CompilerParams(use_tc_tiling_on_sc=False),
    scratch_types=[
        pltpu.VMEM((RPT_ES, K), jnp.int32),
        pltpu.VMEM((RPT_ES, K), jnp.int32),
        pltpu.VMEM((K, 64), F32),
        pltpu.VMEM((K, 64), F32),
        pltpu.VMEM_SHARED((N, 64), F32),
        pltpu.SemaphoreType.DMA,
        pltpu.SemaphoreType.DMA,
    ],
)
def _sc_apply_es(y3, src2, dst2, zeros64, za_out, zb_out,
                 idx_s, idx_d, buf_a, buf_b, acc, sem_a, sem_b):
    c = lax.axis_index("c")
    s = lax.axis_index("s")
    w = s * NC + c
    pltpu.sync_copy(src2.at[pl.ds(w * RPT_ES, RPT_ES)], idx_s)
    pltpu.sync_copy(dst2.at[pl.ds(w * RPT_ES, RPT_ES)], idx_d)

    @pl.when(s < NWR)
    def _():
        pltpu.sync_copy(zeros64, buf_a)
        for k in range(WPT // K):
            pltpu.sync_copy(buf_a, acc.at[pl.ds(s * WPT + k * K, K)])

    plsc.subcore_barrier()

    def it(j, carry):
        r = 2 * j
        d0 = pltpu.async_copy(y3.at[idx_s.at[r]], buf_a, sem_a)
        d1 = pltpu.async_copy(y3.at[idx_s.at[r + 1]], buf_b, sem_b)
        d0.wait()
        pltpu.sync_copy(buf_a, acc.at[idx_d.at[r]], add=True)
        d1.wait()
        pltpu.sync_copy(buf_b, acc.at[idx_d.at[r + 1]], add=True)
        return carry

    lax.fori_loop(0, RPT_ES // 2, it, 0)
    plsc.subcore_barrier()

    @pl.when((c == 0) & (s < NWR))
    def _():
        pltpu.sync_copy(acc.at[pl.ds(s * WPT, WPT)],
                        za_out.at[pl.ds(s * WPT, WPT)])

    @pl.when((c == 1) & (s < NWR))
    def _():
        pltpu.sync_copy(acc.at[pl.ds(s * WPT, WPT)],
                        zb_out.at[pl.ds(s * WPT, WPT)])


_GRID = 10
_RB = N // _GRID


def _norm_of(d0_ref, d1_ref):
    deg = d0_ref[:, :1] + d1_ref[:, :1]
    return lax.rsqrt(jnp.maximum(deg, 1.0))


def _tc_prescale_body(d0_ref, d1_ref, x_ref, y0_ref, y1_ref):
    norm = _norm_of(d0_ref, d1_ref)
    xs = x_ref[...] * norm
    y0_ref[...] = xs[:, :128]
    y1_ref[...] = xs[:, 128:]


_tc_prescale = pl.pallas_call(
    _tc_prescale_body,
    grid=(_GRID,),
    in_specs=[
        pl.BlockSpec((_RB, 16), lambda i: (i, 0)),
        pl.BlockSpec((_RB, 16), lambda i: (i, 0)),
        pl.BlockSpec((_RB, D), lambda i: (i, 0)),
    ],
    out_specs=[
        pl.BlockSpec((_RB, 128), lambda i: (i, 0)),
        pl.BlockSpec((_RB, 128), lambda i: (i, 0)),
    ],
    out_shape=[jax.ShapeDtypeStruct((N, 128), F32)] * 2,
)


def _tc_glue_body(z0_ref, z1_ref, d0_ref, d1_ref, w_ref, b_ref,
                  emb_ref, y0_ref, y1_ref):
    norm = _norm_of(d0_ref, d1_ref)
    z = jnp.concatenate([z0_ref[...], z1_ref[...]], axis=1) * norm
    h = jnp.dot(z, w_ref[...], preferred_element_type=F32) + b_ref[...]
    emb_ref[...] = h
    y = jnp.maximum(h, 0.0) * norm
    y0_ref[...] = y[:, :128]
    y1_ref[...] = y[:, 128:]


_tc_glue = pl.pallas_call(
    _tc_glue_body,
    grid=(_GRID,),
    in_specs=[
        pl.BlockSpec((_RB, 128), lambda i: (i, 0)),
        pl.BlockSpec((_RB, 128), lambda i: (i, 0)),
        pl.BlockSpec((_RB, 16), lambda i: (i, 0)),
        pl.BlockSpec((_RB, 16), lambda i: (i, 0)),
        pl.BlockSpec((D, D), lambda i: (0, 0)),
        pl.BlockSpec((1, D), lambda i: (0, 0)),
    ],
    out_specs=[
        pl.BlockSpec((_RB, D), lambda i: (i, 0)),
        pl.BlockSpec((_RB, 128), lambda i: (i, 0)),
        pl.BlockSpec((_RB, 128), lambda i: (i, 0)),
    ],
    out_shape=[
        jax.ShapeDtypeStruct((N, D), F32),
        jax.ShapeDtypeStruct((N, 128), F32),
        jax.ShapeDtypeStruct((N, 128), F32),
    ],
)


def _tc_glue3_body(z0_ref, z1_ref, d0_ref, d1_ref, w_ref, b_ref, w3_ref,
                   emb_ref, y3_ref):
    norm = _norm_of(d0_ref, d1_ref)
    z = jnp.concatenate([z0_ref[...], z1_ref[...]], axis=1) * norm
    h = jnp.dot(z, w_ref[...], preferred_element_type=F32) + b_ref[...]
    emb_ref[...] = h
    y = jnp.maximum(h, 0.0) * norm
    y3_ref[...] = jnp.dot(y, w3_ref[...], preferred_element_type=F32)


_tc_glue3 = pl.pallas_call(
    _tc_glue3_body,
    grid=(_GRID,),
    in_specs=[
        pl.BlockSpec((_RB, 128), lambda i: (i, 0)),
        pl.BlockSpec((_RB, 128), lambda i: (i, 0)),
        pl.BlockSpec((_RB, 16), lambda i: (i, 0)),
        pl.BlockSpec((_RB, 16), lambda i: (i, 0)),
        pl.BlockSpec((D, D), lambda i: (0, 0)),
        pl.BlockSpec((1, D), lambda i: (0, 0)),
        pl.BlockSpec((D, 64), lambda i: (0, 0)),
    ],
    out_specs=[
        pl.BlockSpec((_RB, D), lambda i: (i, 0)),
        pl.BlockSpec((_RB, 64), lambda i: (i, 0)),
    ],
    out_shape=[
        jax.ShapeDtypeStruct((N, D), F32),
        jax.ShapeDtypeStruct((N, 64), F32),
    ],
)


def _tc_final_body(za_ref, zb_ref, d0_ref, d1_ref, b3_ref, out_ref, emb_ref):
    norm = _norm_of(d0_ref, d1_ref)
    z = (za_ref[...] + zb_ref[...]) * norm
    h = z[:, :NCLS] + b3_ref[...]
    emb_ref[...] = h
    m = jnp.max(h, axis=1, keepdims=True)
    lse = jnp.log(jnp.sum(jnp.exp(h - m), axis=1, keepdims=True)) + m
    out_ref[...] = h - lse


_tc_final = pl.pallas_call(
    _tc_final_body,
    grid=(_GRID,),
    in_specs=[
        pl.BlockSpec((_RB, 64), lambda i: (i, 0)),
        pl.BlockSpec((_RB, 64), lambda i: (i, 0)),
        pl.BlockSpec((_RB, 16), lambda i: (i, 0)),
        pl.BlockSpec((_RB, 16), lambda i: (i, 0)),
        pl.BlockSpec((1, NCLS), lambda i: (0, 0)),
    ],
    out_specs=[
        pl.BlockSpec((_RB, NCLS), lambda i: (i, 0)),
        pl.BlockSpec((_RB, NCLS), lambda i: (i, 0)),
    ],
    out_shape=[
        jax.ShapeDtypeStruct((N, NCLS), F32),
        jax.ShapeDtypeStruct((N, NCLS), F32),
    ],
)


def kernel(x, edge_index, W1, b1, W2, b2, W3, b3):
    src2 = edge_index[0].reshape(ROWS, K)
    dst2 = edge_index[1].reshape(ROWS, K)
    ones8 = jnp.ones((K, 16), F32)
    zeros8 = jnp.zeros((WPT, 16), F32)
    zeros128 = jnp.zeros((K, 128), F32)
    zeros64 = jnp.zeros((K, 64), F32)
    W3p = jnp.zeros((D, 64), F32).at[:, :NCLS].set(W3)
    b1r = b1.reshape(1, D)
    b2r = b2.reshape(1, D)
    b3r = b3.reshape(1, NCLS)

    d0, d1 = _sc_deg(dst2, ones8, zeros8)
    y0, y1 = _tc_prescale(d0, d1, x)
    z0, z1 = _sc_apply_fs(y0, y1, src2, dst2, zeros128)
    emb1, y20, y21 = _tc_glue(z0, z1, d0, d1, W1, b1r)
    z20, z21 = _sc_apply_fs(y20, y21, src2, dst2, zeros128)
    emb2, y3 = _tc_glue3(z20, z21, d0, d1, W2, b2r, W3p)
    za, zb = _sc_apply_es(y3, src2, dst2, zeros64)
    out, emb3 = _tc_final(za, zb, d0, d1, b3r)
    return (out, emb1, emb2, emb3)


# speedup vs baseline: 11.9913x; 1.0267x over previous
import functools

import jax
import jax.numpy as jnp
from jax import lax
from jax.experimental import pallas as pl
from jax.experimental.pallas import tpu as pltpu
from jax.experimental.pallas import tpu_sc as plsc

N = 10000
E = 160000
D = 256
NCLS = 40
NC, NS = 2, 16
K = 125
ROWS = E // K
RPT_FS = ROWS // NS
RPT_ES = ROWS // (NC * NS)
NWR = 10
WPT = N // NWR

_MESH = plsc.VectorSubcoreMesh(
    core_axis_name="c", subcore_axis_name="s", num_cores=NC, num_subcores=NS)

F32 = jnp.float32


@functools.partial(
    pl.kernel,
    out_type=(jax.ShapeDtypeStruct((N, 16), F32),) * 2,
    mesh=_MESH,
    compiler_params=pltpu.CompilerParams(use_tc_tiling_on_sc=False),
    scratch_types=[
        pltpu.VMEM((RPT_ES, K), jnp.int32),
        pltpu.VMEM((K, 16), F32),
        pltpu.VMEM((WPT, 16), F32),
        pltpu.VMEM_SHARED((N, 16), F32),
    ],
)
def _sc_deg(dst2, ones8, zeros8, d0_out, d1_out, idx_d, ones_v, zbuf, degsh):
    c = lax.axis_index("c")
    s = lax.axis_index("s")
    w = s * NC + c
    pltpu.sync_copy(dst2.at[pl.ds(w * RPT_ES, RPT_ES)], idx_d)
    pltpu.sync_copy(ones8, ones_v)

    @pl.when(s < NWR)
    def _():
        pltpu.sync_copy(zeros8, zbuf)
        pltpu.sync_copy(zbuf, degsh.at[pl.ds(s * WPT, WPT)])

    plsc.subcore_barrier()

    def it(j, carry):
        pltpu.sync_copy(ones_v, degsh.at[idx_d.at[j]], add=True)
        return carry

    lax.fori_loop(0, RPT_ES, it, 0)
    plsc.subcore_barrier()

    @pl.when((c == 0) & (s < NWR))
    def _():
        pltpu.sync_copy(degsh.at[pl.ds(s * WPT, WPT)],
                        d0_out.at[pl.ds(s * WPT, WPT)])

    @pl.when((c == 1) & (s < NWR))
    def _():
        pltpu.sync_copy(degsh.at[pl.ds(s * WPT, WPT)],
                        d1_out.at[pl.ds(s * WPT, WPT)])


@functools.partial(
    pl.kernel,
    out_type=(jax.ShapeDtypeStruct((N, 128), F32),) * 2,
    mesh=_MESH,
    compiler_params=pltpu.CompilerParams(use_tc_tiling_on_sc=False),
    scratch_types=[
        pltpu.VMEM((RPT_FS // 2, K), jnp.int32),
        pltpu.VMEM((RPT_FS // 2, K), jnp.int32),
        pltpu.VMEM((K, 128), F32),
        pltpu.VMEM((K, 128), F32),
        pltpu.VMEM_SHARED((N, 128), F32),
        pltpu.SemaphoreType.DMA,
        pltpu.SemaphoreType.DMA,
        pltpu.SemaphoreType.DMA,
        pltpu.SemaphoreType.DMA,
    ],
)
def _sc_apply_fs(y0, y1, src2, dst2, zeros128, z0_out, z1_out,
                 idx_s, idx_d, buf_a, buf_b, acc, sem_a, sem_b,
                 ssem_a, ssem_b):
    c = lax.axis_index("c")
    s = lax.axis_index("s")
    half = RPT_FS // 2

    @pl.when(s < NWR)
    def _():
        pltpu.sync_copy(zeros128, buf_a)
        for k in range(WPT // K):
            pltpu.sync_copy(buf_a, acc.at[pl.ds(s * WPT + k * K, K)])

    plsc.subcore_barrier()

    def run(y):
        for p in range(2):
            pltpu.sync_copy(src2.at[pl.ds(s * RPT_FS + p * half, half)], idx_s)
            pltpu.sync_copy(dst2.at[pl.ds(s * RPT_FS + p * half, half)], idx_d)

            def it(j, carry):
                r = 2 * j

                @pl.when(j > 0)
                def _():
                    pltpu.make_async_copy(buf_a, acc.at[idx_d.at[0]],
                                          ssem_a).wait()

                ga = pltpu.async_copy(y.at[idx_s.at[r]], buf_a, sem_a)

                @pl.when(j > 0)
                def _():
                    pltpu.make_async_copy(buf_b, acc.at[idx_d.at[0]],
                                          ssem_b).wait()

                gb = pltpu.async_copy(y.at[idx_s.at[r + 1]], buf_b, sem_b)
                ga.wait()
                pltpu.async_copy(buf_a, acc.at[idx_d.at[r]], ssem_a, add=True)
                gb.wait()
                pltpu.async_copy(buf_b, acc.at[idx_d.at[r + 1]], ssem_b,
                                 add=True)
                return carry

            lax.fori_loop(0, half // 2, it, 0)
            pltpu.make_async_copy(buf_a, acc.at[idx_d.at[0]], ssem_a).wait()
            pltpu.make_async_copy(buf_b, acc.at[idx_d.at[0]], ssem_b).wait()

    @pl.when(c == 0)
    def _():
        run(y0)

    @pl.when(c == 1)
    def _():
        run(y1)

    plsc.subcore_barrier()

    @pl.when((c == 0) & (s < NWR))
    def _():
        pltpu.sync_copy(acc.at[pl.ds(s * WPT, WPT)],
                        z0_out.at[pl.ds(s * WPT, WPT)])

    @pl.when((c == 1) & (s < NWR))
    def _():
        pltpu.sync_copy(acc.at[pl.ds(s * WPT, WPT)],
                        z1_out.at[pl.ds(s * WPT, WPT)])


@functools.partial(
    pl.kernel,
    out_type=(jax.ShapeDtypeStruct((N, 64), F32),) * 2,
    mesh=_MESH,
    compiler_params=pltpu.CompilerParams(use_tc_tiling_on_sc=False),
    scratch_types=[
        pltpu.VMEM((RPT_ES, K), jnp.int32),
        pltpu.VMEM((RPT_ES, K), jnp.int32),
        pltpu.VMEM((K, 64), F32),
        pltpu.VMEM((K, 64), F32),
        pltpu.VMEM_SHARED((N, 64), F32),
        pltpu.SemaphoreType.DMA,
        pltpu.SemaphoreType.DMA,
        pltpu.SemaphoreType.DMA,
        pltpu.SemaphoreType.DMA,
    ],
)
def _sc_apply_es(y3, src2, dst2, zeros64, za_out, zb_out,
                 idx_s, idx_d, buf_a, buf_b, acc, sem_a, sem_b,
                 ssem_a, ssem_b):
    c = lax.axis_index("c")
    s = lax.axis_index("s")
    w = s * NC + c
    pltpu.sync_copy(src2.at[pl.ds(w * RPT_ES, RPT_ES)], idx_s)
    pltpu.sync_copy(dst2.at[pl.ds(w * RPT_ES, RPT_ES)], idx_d)

    @pl.when(s < NWR)
    def _():
        pltpu.sync_copy(zeros64, buf_a)
        for k in range(WPT // K):
            pltpu.sync_copy(buf_a, acc.at[pl.ds(s * WPT + k * K, K)])

    plsc.subcore_barrier()

    def it(j, carry):
        r = 2 * j

        @pl.when(j > 0)
        def _():
            pltpu.make_async_copy(buf_a, acc.at[idx_d.at[0]], ssem_a).wait()

        ga = pltpu.async_copy(y3.at[idx_s.at[r]], buf_a, sem_a)

        @pl.when(j > 0)
        def _():
            pltpu.make_async_copy(buf_b, acc.at[idx_d.at[0]], ssem_b).wait()

        gb = pltpu.async_copy(y3.at[idx_s.at[r + 1]], buf_b, sem_b)
        ga.wait()
        pltpu.async_copy(buf_a, acc.at[idx_d.at[r]], ssem_a, add=True)
        gb.wait()
        pltpu.async_copy(buf_b, acc.at[idx_d.at[r + 1]], ssem_b, add=True)
        return carry

    lax.fori_loop(0, RPT_ES // 2, it, 0)
    pltpu.make_async_copy(buf_a, acc.at[idx_d.at[0]], ssem_a).wait()
    pltpu.make_async_copy(buf_b, acc.at[idx_d.at[0]], ssem_b).wait()
    plsc.subcore_barrier()

    @pl.when((c == 0) & (s < NWR))
    def _():
        pltpu.sync_copy(acc.at[pl.ds(s * WPT, WPT)],
                        za_out.at[pl.ds(s * WPT, WPT)])

    @pl.when((c == 1) & (s < NWR))
    def _():
        pltpu.sync_copy(acc.at[pl.ds(s * WPT, WPT)],
                        zb_out.at[pl.ds(s * WPT, WPT)])


_GRID = 10
_RB = N // _GRID


def _norm_of(d0_ref, d1_ref):
    deg = d0_ref[:, :1] + d1_ref[:, :1]
    return lax.rsqrt(jnp.maximum(deg, 1.0))


def _tc_prescale_body(d0_ref, d1_ref, x_ref, y0_ref, y1_ref):
    norm = _norm_of(d0_ref, d1_ref)
    xs = x_ref[...] * norm
    y0_ref[...] = xs[:, :128]
    y1_ref[...] = xs[:, 128:]


_tc_prescale = pl.pallas_call(
    _tc_prescale_body,
    grid=(_GRID,),
    in_specs=[
        pl.BlockSpec((_RB, 16), lambda i: (i, 0)),
        pl.BlockSpec((_RB, 16), lambda i: (i, 0)),
        pl.BlockSpec((_RB, D), lambda i: (i, 0)),
    ],
    out_specs=[
        pl.BlockSpec((_RB, 128), lambda i: (i, 0)),
        pl.BlockSpec((_RB, 128), lambda i: (i, 0)),
    ],
    out_shape=[jax.ShapeDtypeStruct((N, 128), F32)] * 2,
)


def _tc_glue_body(z0_ref, z1_ref, d0_ref, d1_ref, w_ref, b_ref,
                  emb_ref, y0_ref, y1_ref):
    norm = _norm_of(d0_ref, d1_ref)
    z = jnp.concatenate([z0_ref[...], z1_ref[...]], axis=1) * norm
    h = jnp.dot(z, w_ref[...], preferred_element_type=F32) + b_ref[...]
    emb_ref[...] = h
    y = jnp.maximum(h, 0.0) * norm
    y0_ref[...] = y[:, :128]
    y1_ref[...] = y[:, 128:]


_tc_glue = pl.pallas_call(
    _tc_glue_body,
    grid=(_GRID,),
    in_specs=[
        pl.BlockSpec((_RB, 128), lambda i: (i, 0)),
        pl.BlockSpec((_RB, 128), lambda i: (i, 0)),
        pl.BlockSpec((_RB, 16), lambda i: (i, 0)),
        pl.BlockSpec((_RB, 16), lambda i: (i, 0)),
        pl.BlockSpec((D, D), lambda i: (0, 0)),
        pl.BlockSpec((1, D), lambda i: (0, 0)),
    ],
    out_specs=[
        pl.BlockSpec((_RB, D), lambda i: (i, 0)),
        pl.BlockSpec((_RB, 128), lambda i: (i, 0)),
        pl.BlockSpec((_RB, 128), lambda i: (i, 0)),
    ],
    out_shape=[
        jax.ShapeDtypeStruct((N, D), F32),
        jax.ShapeDtypeStruct((N, 128), F32),
        jax.ShapeDtypeStruct((N, 128), F32),
    ],
)


def _tc_glue3_body(z0_ref, z1_ref, d0_ref, d1_ref, w_ref, b_ref, w3_ref,
                   emb_ref, y3_ref):
    norm = _norm_of(d0_ref, d1_ref)
    z = jnp.concatenate([z0_ref[...], z1_ref[...]], axis=1) * norm
    h = jnp.dot(z, w_ref[...], preferred_element_type=F32) + b_ref[...]
    emb_ref[...] = h
    y = jnp.maximum(h, 0.0) * norm
    y3_ref[...] = jnp.dot(y, w3_ref[...], preferred_element_type=F32)


_tc_glue3 = pl.pallas_call(
    _tc_glue3_body,
    grid=(_GRID,),
    in_specs=[
        pl.BlockSpec((_RB, 128), lambda i: (i, 0)),
        pl.BlockSpec((_RB, 128), lambda i: (i, 0)),
        pl.BlockSpec((_RB, 16), lambda i: (i, 0)),
        pl.BlockSpec((_RB, 16), lambda i: (i, 0)),
        pl.BlockSpec((D, D), lambda i: (0, 0)),
        pl.BlockSpec((1, D), lambda i: (0, 0)),
        pl.BlockSpec((D, 64), lambda i: (0, 0)),
    ],
    out_specs=[
        pl.BlockSpec((_RB, D), lambda i: (i, 0)),
        pl.BlockSpec((_RB, 64), lambda i: (i, 0)),
    ],
    out_shape=[
        jax.ShapeDtypeStruct((N, D), F32),
        jax.ShapeDtypeStruct((N, 64), F32),
    ],
)


def _tc_final_body(za_ref, zb_ref, d0_ref, d1_ref, b3_ref, out_ref, emb_ref):
    norm = _norm_of(d0_ref, d1_ref)
    z = (za_ref[...] + zb_ref[...]) * norm
    h = z[:, :NCLS] + b3_ref[...]
    emb_ref[...] = h
    m = jnp.max(h, axis=1, keepdims=True)
    lse = jnp.log(jnp.sum(jnp.exp(h - m), axis=1, keepdims=True)) + m
    out_ref[...] = h - lse


_tc_final = pl.pallas_call(
    _tc_final_body,
    grid=(_GRID,),
    in_specs=[
        pl.BlockSpec((_RB, 64), lambda i: (i, 0)),
        pl.BlockSpec((_RB, 64), lambda i: (i, 0)),
        pl.BlockSpec((_RB, 16), lambda i: (i, 0)),
        pl.BlockSpec((_RB, 16), lambda i: (i, 0)),
        pl.BlockSpec((1, NCLS), lambda i: (0, 0)),
    ],
    out_specs=[
        pl.BlockSpec((_RB, NCLS), lambda i: (i, 0)),
        pl.BlockSpec((_RB, NCLS), lambda i: (i, 0)),
    ],
    out_shape=[
        jax.ShapeDtypeStruct((N, NCLS), F32),
        jax.ShapeDtypeStruct((N, NCLS), F32),
    ],
)


def kernel(x, edge_index, W1, b1, W2, b2, W3, b3):
    src2 = edge_index[0].reshape(ROWS, K)
    dst2 = edge_index[1].reshape(ROWS, K)
    ones8 = jnp.ones((K, 16), F32)
    zeros8 = jnp.zeros((WPT, 16), F32)
    zeros128 = jnp.zeros((K, 128), F32)
    zeros64 = jnp.zeros((K, 64), F32)
    W3p = jnp.zeros((D, 64), F32).at[:, :NCLS].set(W3)
    b1r = b1.reshape(1, D)
    b2r = b2.reshape(1, D)
    b3r = b3.reshape(1, NCLS)

    d0, d1 = _sc_deg(dst2, ones8, zeros8)
    y0, y1 = _tc_prescale(d0, d1, x)
    z0, z1 = _sc_apply_fs(y0, y1, src2, dst2, zeros128)
    emb1, y20, y21 = _tc_glue(z0, z1, d0, d1, W1, b1r)
    z20, z21 = _sc_apply_fs(y20, y21, src2, dst2, zeros128)
    emb2, y3 = _tc_glue3(z20, z21, d0, d1, W2, b2r, W3p)
    za, zb = _sc_apply_es(y3, src2, dst2, zeros64)
    out, emb3 = _tc_final(za, zb, d0, d1, b3r)
    return (out, emb1, emb2, emb3)


# speedup vs baseline: 13.6100x; 1.1350x over previous
import functools

import jax
import jax.numpy as jnp
from jax import lax
from jax.experimental import pallas as pl
from jax.experimental.pallas import tpu as pltpu
from jax.experimental.pallas import tpu_sc as plsc

N = 10000
E = 160000
D = 256
NCLS = 40
NC, NS = 2, 16
K = 125
ROWS = E // K
RPT_FS = ROWS // NS
RPT_ES = ROWS // (NC * NS)
NWR = 10
WPT = N // NWR

_MESH = plsc.VectorSubcoreMesh(
    core_axis_name="c", subcore_axis_name="s", num_cores=NC, num_subcores=NS)

F32 = jnp.float32


@functools.partial(
    pl.kernel,
    out_type=(jax.ShapeDtypeStruct((N, 16), F32),) * 2,
    mesh=_MESH,
    compiler_params=pltpu.CompilerParams(use_tc_tiling_on_sc=False),
    scratch_types=[
        pltpu.VMEM((RPT_ES, K), jnp.int32),
        pltpu.VMEM((K, 16), F32),
        pltpu.VMEM((WPT, 16), F32),
        pltpu.VMEM_SHARED((N, 16), F32),
    ],
)
def _sc_deg(dst2, ones8, zeros8, d0_out, d1_out, idx_d, ones_v, zbuf, degsh):
    c = lax.axis_index("c")
    s = lax.axis_index("s")
    w = s * NC + c
    pltpu.sync_copy(dst2.at[pl.ds(w * RPT_ES, RPT_ES)], idx_d)
    pltpu.sync_copy(ones8, ones_v)

    @pl.when(s < NWR)
    def _():
        pltpu.sync_copy(zeros8, zbuf)
        pltpu.sync_copy(zbuf, degsh.at[pl.ds(s * WPT, WPT)])

    plsc.subcore_barrier()

    def it(j, carry):
        pltpu.sync_copy(ones_v, degsh.at[idx_d.at[j]], add=True)
        return carry

    lax.fori_loop(0, RPT_ES, it, 0)
    plsc.subcore_barrier()

    @pl.when((c == 0) & (s < NWR))
    def _():
        pltpu.sync_copy(degsh.at[pl.ds(s * WPT, WPT)],
                        d0_out.at[pl.ds(s * WPT, WPT)])

    @pl.when((c == 1) & (s < NWR))
    def _():
        pltpu.sync_copy(degsh.at[pl.ds(s * WPT, WPT)],
                        d1_out.at[pl.ds(s * WPT, WPT)])


@functools.partial(
    pl.kernel,
    out_type=(jax.ShapeDtypeStruct((N, 128), jnp.bfloat16),) * 2,
    mesh=_MESH,
    compiler_params=pltpu.CompilerParams(use_tc_tiling_on_sc=False),
    scratch_types=[
        pltpu.VMEM((RPT_FS // 2, K), jnp.int32),
        pltpu.VMEM((RPT_FS // 2, K), jnp.int32),
        pltpu.VMEM((K, 128), jnp.bfloat16),
        pltpu.VMEM((K, 128), jnp.bfloat16),
        pltpu.VMEM_SHARED((N, 128), jnp.bfloat16),
        pltpu.SemaphoreType.DMA,
        pltpu.SemaphoreType.DMA,
        pltpu.SemaphoreType.DMA,
        pltpu.SemaphoreType.DMA,
    ],
)
def _sc_apply_fs(y0, y1, src2, dst2, zeros128, z0_out, z1_out,
                 idx_s, idx_d, buf_a, buf_b, acc, sem_a, sem_b,
                 ssem_a, ssem_b):
    c = lax.axis_index("c")
    s = lax.axis_index("s")
    half = RPT_FS // 2

    @pl.when(s < NWR)
    def _():
        pltpu.sync_copy(zeros128, buf_a)
        for k in range(WPT // K):
            pltpu.sync_copy(buf_a, acc.at[pl.ds(s * WPT + k * K, K)])

    plsc.subcore_barrier()

    def run(y):
        for p in range(2):
            pltpu.sync_copy(src2.at[pl.ds(s * RPT_FS + p * half, half)], idx_s)
            pltpu.sync_copy(dst2.at[pl.ds(s * RPT_FS + p * half, half)], idx_d)

            def it(j, carry):
                r = 2 * j

                @pl.when(j > 0)
                def _():
                    pltpu.make_async_copy(buf_a, acc.at[idx_d.at[0]],
                                          ssem_a).wait()

                ga = pltpu.async_copy(y.at[idx_s.at[r]], buf_a, sem_a)

                @pl.when(j > 0)
                def _():
                    pltpu.make_async_copy(buf_b, acc.at[idx_d.at[0]],
                                          ssem_b).wait()

                gb = pltpu.async_copy(y.at[idx_s.at[r + 1]], buf_b, sem_b)
                ga.wait()
                pltpu.async_copy(buf_a, acc.at[idx_d.at[r]], ssem_a, add=True)
                gb.wait()
                pltpu.async_copy(buf_b, acc.at[idx_d.at[r + 1]], ssem_b,
                                 add=True)
                return carry

            lax.fori_loop(0, half // 2, it, 0)
            pltpu.make_async_copy(buf_a, acc.at[idx_d.at[0]], ssem_a).wait()
            pltpu.make_async_copy(buf_b, acc.at[idx_d.at[0]], ssem_b).wait()

    @pl.when(c == 0)
    def _():
        run(y0)

    @pl.when(c == 1)
    def _():
        run(y1)

    plsc.subcore_barrier()

    @pl.when((c == 0) & (s < NWR))
    def _():
        pltpu.sync_copy(acc.at[pl.ds(s * WPT, WPT)],
                        z0_out.at[pl.ds(s * WPT, WPT)])

    @pl.when((c == 1) & (s < NWR))
    def _():
        pltpu.sync_copy(acc.at[pl.ds(s * WPT, WPT)],
                        z1_out.at[pl.ds(s * WPT, WPT)])


@functools.partial(
    pl.kernel,
    out_type=(jax.ShapeDtypeStruct((N, 64), F32),) * 2,
    mesh=_MESH,
    compiler_params=pltpu.CompilerParams(use_tc_tiling_on_sc=False),
    scratch_types=[
        pltpu.VMEM((RPT_ES, K), jnp.int32),
        pltpu.VMEM((RPT_ES, K), jnp.int32),
        pltpu.VMEM((K, 64), F32),
        pltpu.VMEM((K, 64), F32),
        pltpu.VMEM_SHARED((N, 64), F32),
        pltpu.SemaphoreType.DMA,
        pltpu.SemaphoreType.DMA,
        pltpu.SemaphoreType.DMA,
        pltpu.SemaphoreType.DMA,
    ],
)
def _sc_apply_es(y3, src2, dst2, zeros64, za_out, zb_out,
                 idx_s, idx_d, buf_a, buf_b, acc, sem_a, sem_b,
                 ssem_a, ssem_b):
    c = lax.axis_index("c")
    s = lax.axis_index("s")
    w = s * NC + c
    pltpu.sync_copy(src2.at[pl.ds(w * RPT_ES, RPT_ES)], idx_s)
    pltpu.sync_copy(dst2.at[pl.ds(w * RPT_ES, RPT_ES)], idx_d)

    @pl.when(s < NWR)
    def _():
        pltpu.sync_copy(zeros64, buf_a)
        for k in range(WPT // K):
            pltpu.sync_copy(buf_a, acc.at[pl.ds(s * WPT + k * K, K)])

    plsc.subcore_barrier()

    def it(j, carry):
        r = 2 * j

        @pl.when(j > 0)
        def _():
            pltpu.make_async_copy(buf_a, acc.at[idx_d.at[0]], ssem_a).wait()

        ga = pltpu.async_copy(y3.at[idx_s.at[r]], buf_a, sem_a)

        @pl.when(j > 0)
        def _():
            pltpu.make_async_copy(buf_b, acc.at[idx_d.at[0]], ssem_b).wait()

        gb = pltpu.async_copy(y3.at[idx_s.at[r + 1]], buf_b, sem_b)
        ga.wait()
        pltpu.async_copy(buf_a, acc.at[idx_d.at[r]], ssem_a, add=True)
        gb.wait()
        pltpu.async_copy(buf_b, acc.at[idx_d.at[r + 1]], ssem_b, add=True)
        return carry

    lax.fori_loop(0, RPT_ES // 2, it, 0)
    pltpu.make_async_copy(buf_a, acc.at[idx_d.at[0]], ssem_a).wait()
    pltpu.make_async_copy(buf_b, acc.at[idx_d.at[0]], ssem_b).wait()
    plsc.subcore_barrier()

    @pl.when((c == 0) & (s < NWR))
    def _():
        pltpu.sync_copy(acc.at[pl.ds(s * WPT, WPT)],
                        za_out.at[pl.ds(s * WPT, WPT)])

    @pl.when((c == 1) & (s < NWR))
    def _():
        pltpu.sync_copy(acc.at[pl.ds(s * WPT, WPT)],
                        zb_out.at[pl.ds(s * WPT, WPT)])


_GRID = 10
_RB = N // _GRID


def _norm_of(d0_ref, d1_ref):
    deg = d0_ref[:, :1] + d1_ref[:, :1]
    return lax.rsqrt(jnp.maximum(deg, 1.0))


def _tc_prescale_body(d0_ref, d1_ref, x_ref, y0_ref, y1_ref):
    norm = _norm_of(d0_ref, d1_ref)
    xs = (x_ref[...] * norm).astype(jnp.bfloat16)
    y0_ref[...] = xs[:, :128]
    y1_ref[...] = xs[:, 128:]


_tc_prescale = pl.pallas_call(
    _tc_prescale_body,
    grid=(_GRID,),
    in_specs=[
        pl.BlockSpec((_RB, 16), lambda i: (i, 0)),
        pl.BlockSpec((_RB, 16), lambda i: (i, 0)),
        pl.BlockSpec((_RB, D), lambda i: (i, 0)),
    ],
    out_specs=[
        pl.BlockSpec((_RB, 128), lambda i: (i, 0)),
        pl.BlockSpec((_RB, 128), lambda i: (i, 0)),
    ],
    out_shape=[jax.ShapeDtypeStruct((N, 128), jnp.bfloat16)] * 2,
)


def _tc_glue_body(z0_ref, z1_ref, d0_ref, d1_ref, w_ref, b_ref,
                  emb_ref, y0_ref, y1_ref):
    norm = _norm_of(d0_ref, d1_ref)
    z = jnp.concatenate([z0_ref[...], z1_ref[...]],
                        axis=1).astype(F32) * norm
    h = jnp.dot(z, w_ref[...], preferred_element_type=F32) + b_ref[...]
    emb_ref[...] = h
    y = ((jnp.maximum(h, 0.0) * norm)).astype(jnp.bfloat16)
    y0_ref[...] = y[:, :128]
    y1_ref[...] = y[:, 128:]


_tc_glue = pl.pallas_call(
    _tc_glue_body,
    grid=(_GRID,),
    in_specs=[
        pl.BlockSpec((_RB, 128), lambda i: (i, 0)),
        pl.BlockSpec((_RB, 128), lambda i: (i, 0)),
        pl.BlockSpec((_RB, 16), lambda i: (i, 0)),
        pl.BlockSpec((_RB, 16), lambda i: (i, 0)),
        pl.BlockSpec((D, D), lambda i: (0, 0)),
        pl.BlockSpec((1, D), lambda i: (0, 0)),
    ],
    out_specs=[
        pl.BlockSpec((_RB, D), lambda i: (i, 0)),
        pl.BlockSpec((_RB, 128), lambda i: (i, 0)),
        pl.BlockSpec((_RB, 128), lambda i: (i, 0)),
    ],
    out_shape=[
        jax.ShapeDtypeStruct((N, D), F32),
        jax.ShapeDtypeStruct((N, 128), jnp.bfloat16),
        jax.ShapeDtypeStruct((N, 128), jnp.bfloat16),
    ],
)


def _tc_glue3_body(z0_ref, z1_ref, d0_ref, d1_ref, w_ref, b_ref, w3_ref,
                   emb_ref, y3_ref):
    norm = _norm_of(d0_ref, d1_ref)
    z = jnp.concatenate([z0_ref[...], z1_ref[...]],
                        axis=1).astype(F32) * norm
    h = jnp.dot(z, w_ref[...], preferred_element_type=F32) + b_ref[...]
    emb_ref[...] = h
    y = jnp.maximum(h, 0.0) * norm
    y3_ref[...] = jnp.dot(y, w3_ref[...], preferred_element_type=F32)


_tc_glue3 = pl.pallas_call(
    _tc_glue3_body,
    grid=(_GRID,),
    in_specs=[
        pl.BlockSpec((_RB, 128), lambda i: (i, 0)),
        pl.BlockSpec((_RB, 128), lambda i: (i, 0)),
        pl.BlockSpec((_RB, 16), lambda i: (i, 0)),
        pl.BlockSpec((_RB, 16), lambda i: (i, 0)),
        pl.BlockSpec((D, D), lambda i: (0, 0)),
        pl.BlockSpec((1, D), lambda i: (0, 0)),
        pl.BlockSpec((D, 64), lambda i: (0, 0)),
    ],
    out_specs=[
        pl.BlockSpec((_RB, D), lambda i: (i, 0)),
        pl.BlockSpec((_RB, 64), lambda i: (i, 0)),
    ],
    out_shape=[
        jax.ShapeDtypeStruct((N, D), F32),
        jax.ShapeDtypeStruct((N, 64), F32),
    ],
)


def _tc_final_body(za_ref, zb_ref, d0_ref, d1_ref, b3_ref, out_ref, emb_ref):
    norm = _norm_of(d0_ref, d1_ref)
    z = (za_ref[...] + zb_ref[...]) * norm
    h = z[:, :NCLS] + b3_ref[...]
    emb_ref[...] = h
    m = jnp.max(h, axis=1, keepdims=True)
    lse = jnp.log(jnp.sum(jnp.exp(h - m), axis=1, keepdims=True)) + m
    out_ref[...] = h - lse


_tc_final = pl.pallas_call(
    _tc_final_body,
    grid=(_GRID,),
    in_specs=[
        pl.BlockSpec((_RB, 64), lambda i: (i, 0)),
        pl.BlockSpec((_RB, 64), lambda i: (i, 0)),
        pl.BlockSpec((_RB, 16), lambda i: (i, 0)),
        pl.BlockSpec((_RB, 16), lambda i: (i, 0)),
        pl.BlockSpec((1, NCLS), lambda i: (0, 0)),
    ],
    out_specs=[
        pl.BlockSpec((_RB, NCLS), lambda i: (i, 0)),
        pl.BlockSpec((_RB, NCLS), lambda i: (i, 0)),
    ],
    out_shape=[
        jax.ShapeDtypeStruct((N, NCLS), F32),
        jax.ShapeDtypeStruct((N, NCLS), F32),
    ],
)


def kernel(x, edge_index, W1, b1, W2, b2, W3, b3):
    src2 = edge_index[0].reshape(ROWS, K)
    dst2 = edge_index[1].reshape(ROWS, K)
    ones8 = jnp.ones((K, 16), F32)
    zeros8 = jnp.zeros((WPT, 16), F32)
    zeros128 = jnp.zeros((K, 128), jnp.bfloat16)
    zeros64 = jnp.zeros((K, 64), F32)
    W3p = jnp.zeros((D, 64), F32).at[:, :NCLS].set(W3)
    b1r = b1.reshape(1, D)
    b2r = b2.reshape(1, D)
    b3r = b3.reshape(1, NCLS)

    d0, d1 = _sc_deg(dst2, ones8, zeros8)
    y0, y1 = _tc_prescale(d0, d1, x)
    z0, z1 = _sc_apply_fs(y0, y1, src2, dst2, zeros128)
    emb1, y20, y21 = _tc_glue(z0, z1, d0, d1, W1, b1r)
    z20, z21 = _sc_apply_fs(y20, y21, src2, dst2, zeros128)
    emb2, y3 = _tc_glue3(z20, z21, d0, d1, W2, b2r, W3p)
    za, zb = _sc_apply_es(y3, src2, dst2, zeros64)
    out, emb3 = _tc_final(za, zb, d0, d1, b3r)
    return (out, emb1, emb2, emb3)


# speedup vs baseline: 14.0460x; 1.0320x over previous
import functools

import jax
import jax.numpy as jnp
from jax import lax
from jax.experimental import pallas as pl
from jax.experimental.pallas import tpu as pltpu
from jax.experimental.pallas import tpu_sc as plsc

N = 10000
E = 160000
D = 256
NCLS = 40
NC, NS = 2, 16
K = 125
ROWS = E // K
RPT_FS = ROWS // NS
RPT_ES = ROWS // (NC * NS)
NWR = 10
WPT = N // NWR

_MESH = plsc.VectorSubcoreMesh(
    core_axis_name="c", subcore_axis_name="s", num_cores=NC, num_subcores=NS)

F32 = jnp.float32


@functools.partial(
    pl.kernel,
    out_type=(jax.ShapeDtypeStruct((N, 16), F32),) * 2,
    mesh=_MESH,
    compiler_params=pltpu.CompilerParams(use_tc_tiling_on_sc=False),
    scratch_types=[
        pltpu.VMEM((RPT_ES, K), jnp.int32),
        pltpu.VMEM((K, 16), F32),
        pltpu.VMEM((WPT, 16), F32),
        pltpu.VMEM_SHARED((N, 16), F32),
    ],
)
def _sc_deg(dst2, ones8, zeros8, d0_out, d1_out, idx_d, ones_v, zbuf, degsh):
    c = lax.axis_index("c")
    s = lax.axis_index("s")
    w = s * NC + c
    pltpu.sync_copy(dst2.at[pl.ds(w * RPT_ES, RPT_ES)], idx_d)
    pltpu.sync_copy(ones8, ones_v)

    @pl.when(s < NWR)
    def _():
        pltpu.sync_copy(zeros8, zbuf)
        pltpu.sync_copy(zbuf, degsh.at[pl.ds(s * WPT, WPT)])

    plsc.subcore_barrier()

    def it(j, carry):
        pltpu.sync_copy(ones_v, degsh.at[idx_d.at[j]], add=True)
        return carry

    lax.fori_loop(0, RPT_ES, it, 0)
    plsc.subcore_barrier()

    @pl.when((c == 0) & (s < NWR))
    def _():
        pltpu.sync_copy(degsh.at[pl.ds(s * WPT, WPT)],
                        d0_out.at[pl.ds(s * WPT, WPT)])

    @pl.when((c == 1) & (s < NWR))
    def _():
        pltpu.sync_copy(degsh.at[pl.ds(s * WPT, WPT)],
                        d1_out.at[pl.ds(s * WPT, WPT)])


@functools.partial(
    pl.kernel,
    out_type=(jax.ShapeDtypeStruct((N, 128), jnp.bfloat16),) * 2,
    mesh=_MESH,
    compiler_params=pltpu.CompilerParams(use_tc_tiling_on_sc=False),
    scratch_types=[
        pltpu.VMEM((RPT_FS // 2, K), jnp.int32),
        pltpu.VMEM((RPT_FS // 2, K), jnp.int32),
        pltpu.VMEM((K, 128), jnp.bfloat16),
        pltpu.VMEM((K, 128), jnp.bfloat16),
        pltpu.VMEM_SHARED((N, 128), jnp.bfloat16),
        pltpu.SemaphoreType.DMA,
        pltpu.SemaphoreType.DMA,
        pltpu.SemaphoreType.DMA,
        pltpu.SemaphoreType.DMA,
    ],
)
def _sc_apply_fs(y0, y1, src2, dst2, zeros128, z0_out, z1_out,
                 idx_s, idx_d, buf_a, buf_b, acc, sem_a, sem_b,
                 ssem_a, ssem_b):
    c = lax.axis_index("c")
    s = lax.axis_index("s")
    half = RPT_FS // 2

    @pl.when(s < NWR)
    def _():
        pltpu.sync_copy(zeros128, buf_a)
        for k in range(WPT // K):
            pltpu.sync_copy(buf_a, acc.at[pl.ds(s * WPT + k * K, K)])

    plsc.subcore_barrier()

    def run(y):
        for p in range(2):
            pltpu.sync_copy(src2.at[pl.ds(s * RPT_FS + p * half, half)], idx_s)
            pltpu.sync_copy(dst2.at[pl.ds(s * RPT_FS + p * half, half)], idx_d)

            def it(j, carry):
                r = 2 * j

                @pl.when(j > 0)
                def _():
                    pltpu.make_async_copy(buf_a, acc.at[idx_d.at[0]],
                                          ssem_a).wait()

                ga = pltpu.async_copy(y.at[idx_s.at[r]], buf_a, sem_a)

                @pl.when(j > 0)
                def _():
                    pltpu.make_async_copy(buf_b, acc.at[idx_d.at[0]],
                                          ssem_b).wait()

                gb = pltpu.async_copy(y.at[idx_s.at[r + 1]], buf_b, sem_b)
                ga.wait()
                pltpu.async_copy(buf_a, acc.at[idx_d.at[r]], ssem_a, add=True)
                gb.wait()
                pltpu.async_copy(buf_b, acc.at[idx_d.at[r + 1]], ssem_b,
                                 add=True)
                return carry

            lax.fori_loop(0, half // 2, it, 0)
            pltpu.make_async_copy(buf_a, acc.at[idx_d.at[0]], ssem_a).wait()
            pltpu.make_async_copy(buf_b, acc.at[idx_d.at[0]], ssem_b).wait()

    @pl.when(c == 0)
    def _():
        run(y0)

    @pl.when(c == 1)
    def _():
        run(y1)

    plsc.subcore_barrier()

    @pl.when((c == 0) & (s < NWR))
    def _():
        pltpu.sync_copy(acc.at[pl.ds(s * WPT, WPT)],
                        z0_out.at[pl.ds(s * WPT, WPT)])

    @pl.when((c == 1) & (s < NWR))
    def _():
        pltpu.sync_copy(acc.at[pl.ds(s * WPT, WPT)],
                        z1_out.at[pl.ds(s * WPT, WPT)])


@functools.partial(
    pl.kernel,
    out_type=(jax.ShapeDtypeStruct((N, 64), jnp.bfloat16),) * 2,
    mesh=_MESH,
    compiler_params=pltpu.CompilerParams(use_tc_tiling_on_sc=False),
    scratch_types=[
        pltpu.VMEM((RPT_ES, K), jnp.int32),
        pltpu.VMEM((RPT_ES, K), jnp.int32),
        pltpu.VMEM((K, 64), jnp.bfloat16),
        pltpu.VMEM((K, 64), jnp.bfloat16),
        pltpu.VMEM_SHARED((N, 64), jnp.bfloat16),
        pltpu.SemaphoreType.DMA,
        pltpu.SemaphoreType.DMA,
        pltpu.SemaphoreType.DMA,
        pltpu.SemaphoreType.DMA,
    ],
)
def _sc_apply_es(y3, src2, dst2, zeros64, za_out, zb_out,
                 idx_s, idx_d, buf_a, buf_b, acc, sem_a, sem_b,
                 ssem_a, ssem_b):
    c = lax.axis_index("c")
    s = lax.axis_index("s")
    w = s * NC + c
    pltpu.sync_copy(src2.at[pl.ds(w * RPT_ES, RPT_ES)], idx_s)
    pltpu.sync_copy(dst2.at[pl.ds(w * RPT_ES, RPT_ES)], idx_d)

    @pl.when(s < NWR)
    def _():
        pltpu.sync_copy(zeros64, buf_a)
        for k in range(WPT // K):
            pltpu.sync_copy(buf_a, acc.at[pl.ds(s * WPT + k * K, K)])

    plsc.subcore_barrier()

    def it(j, carry):
        r = 2 * j

        @pl.when(j > 0)
        def _():
            pltpu.make_async_copy(buf_a, acc.at[idx_d.at[0]], ssem_a).wait()

        ga = pltpu.async_copy(y3.at[idx_s.at[r]], buf_a, sem_a)

        @pl.when(j > 0)
        def _():
            pltpu.make_async_copy(buf_b, acc.at[idx_d.at[0]], ssem_b).wait()

        gb = pltpu.async_copy(y3.at[idx_s.at[r + 1]], buf_b, sem_b)
        ga.wait()
        pltpu.async_copy(buf_a, acc.at[idx_d.at[r]], ssem_a, add=True)
        gb.wait()
        pltpu.async_copy(buf_b, acc.at[idx_d.at[r + 1]], ssem_b, add=True)
        return carry

    lax.fori_loop(0, RPT_ES // 2, it, 0)
    pltpu.make_async_copy(buf_a, acc.at[idx_d.at[0]], ssem_a).wait()
    pltpu.make_async_copy(buf_b, acc.at[idx_d.at[0]], ssem_b).wait()
    plsc.subcore_barrier()

    @pl.when((c == 0) & (s < NWR))
    def _():
        pltpu.sync_copy(acc.at[pl.ds(s * WPT, WPT)],
                        za_out.at[pl.ds(s * WPT, WPT)])

    @pl.when((c == 1) & (s < NWR))
    def _():
        pltpu.sync_copy(acc.at[pl.ds(s * WPT, WPT)],
                        zb_out.at[pl.ds(s * WPT, WPT)])


_GRID = 10
_RB = N // _GRID


def _norm_of(d0_ref, d1_ref):
    deg = d0_ref[:, :1] + d1_ref[:, :1]
    return lax.rsqrt(jnp.maximum(deg, 1.0))


def _tc_prescale_body(d0_ref, d1_ref, x_ref, y0_ref, y1_ref):
    norm = _norm_of(d0_ref, d1_ref)
    xs = (x_ref[...] * norm).astype(jnp.bfloat16)
    y0_ref[...] = xs[:, :128]
    y1_ref[...] = xs[:, 128:]


_tc_prescale = pl.pallas_call(
    _tc_prescale_body,
    grid=(_GRID,),
    in_specs=[
        pl.BlockSpec((_RB, 16), lambda i: (i, 0)),
        pl.BlockSpec((_RB, 16), lambda i: (i, 0)),
        pl.BlockSpec((_RB, D), lambda i: (i, 0)),
    ],
    out_specs=[
        pl.BlockSpec((_RB, 128), lambda i: (i, 0)),
        pl.BlockSpec((_RB, 128), lambda i: (i, 0)),
    ],
    out_shape=[jax.ShapeDtypeStruct((N, 128), jnp.bfloat16)] * 2,
)


def _tc_glue_body(z0_ref, z1_ref, d0_ref, d1_ref, w_ref, b_ref,
                  emb_ref, y0_ref, y1_ref):
    norm = _norm_of(d0_ref, d1_ref)
    z = jnp.concatenate([z0_ref[...], z1_ref[...]],
                        axis=1).astype(F32) * norm
    h = jnp.dot(z, w_ref[...], preferred_element_type=F32) + b_ref[...]
    emb_ref[...] = h
    y = ((jnp.maximum(h, 0.0) * norm)).astype(jnp.bfloat16)
    y0_ref[...] = y[:, :128]
    y1_ref[...] = y[:, 128:]


_tc_glue = pl.pallas_call(
    _tc_glue_body,
    grid=(_GRID,),
    in_specs=[
        pl.BlockSpec((_RB, 128), lambda i: (i, 0)),
        pl.BlockSpec((_RB, 128), lambda i: (i, 0)),
        pl.BlockSpec((_RB, 16), lambda i: (i, 0)),
        pl.BlockSpec((_RB, 16), lambda i: (i, 0)),
        pl.BlockSpec((D, D), lambda i: (0, 0)),
        pl.BlockSpec((1, D), lambda i: (0, 0)),
    ],
    out_specs=[
        pl.BlockSpec((_RB, D), lambda i: (i, 0)),
        pl.BlockSpec((_RB, 128), lambda i: (i, 0)),
        pl.BlockSpec((_RB, 128), lambda i: (i, 0)),
    ],
    out_shape=[
        jax.ShapeDtypeStruct((N, D), F32),
        jax.ShapeDtypeStruct((N, 128), jnp.bfloat16),
        jax.ShapeDtypeStruct((N, 128), jnp.bfloat16),
    ],
)


def _tc_glue3_body(z0_ref, z1_ref, d0_ref, d1_ref, w_ref, b_ref, w3_ref,
                   emb_ref, y3_ref):
    norm = _norm_of(d0_ref, d1_ref)
    z = jnp.concatenate([z0_ref[...], z1_ref[...]],
                        axis=1).astype(F32) * norm
    h = jnp.dot(z, w_ref[...], preferred_element_type=F32) + b_ref[...]
    emb_ref[...] = h
    y = jnp.maximum(h, 0.0) * norm
    y3_ref[...] = jnp.dot(y, w3_ref[...],
                          preferred_element_type=F32).astype(jnp.bfloat16)


_tc_glue3 = pl.pallas_call(
    _tc_glue3_body,
    grid=(_GRID,),
    in_specs=[
        pl.BlockSpec((_RB, 128), lambda i: (i, 0)),
        pl.BlockSpec((_RB, 128), lambda i: (i, 0)),
        pl.BlockSpec((_RB, 16), lambda i: (i, 0)),
        pl.BlockSpec((_RB, 16), lambda i: (i, 0)),
        pl.BlockSpec((D, D), lambda i: (0, 0)),
        pl.BlockSpec((1, D), lambda i: (0, 0)),
        pl.BlockSpec((D, 64), lambda i: (0, 0)),
    ],
    out_specs=[
        pl.BlockSpec((_RB, D), lambda i: (i, 0)),
        pl.BlockSpec((_RB, 64), lambda i: (i, 0)),
    ],
    out_shape=[
        jax.ShapeDtypeStruct((N, D), F32),
        jax.ShapeDtypeStruct((N, 64), jnp.bfloat16),
    ],
)


def _tc_final_body(za_ref, zb_ref, d0_ref, d1_ref, b3_ref, out_ref, emb_ref):
    norm = _norm_of(d0_ref, d1_ref)
    z = (za_ref[...].astype(F32) + zb_ref[...].astype(F32)) * norm
    h = z[:, :NCLS] + b3_ref[...]
    emb_ref[...] = h
    m = jnp.max(h, axis=1, keepdims=True)
    lse = jnp.log(jnp.sum(jnp.exp(h - m), axis=1, keepdims=True)) + m
    out_ref[...] = h - lse


_tc_final = pl.pallas_call(
    _tc_final_body,
    grid=(_GRID,),
    in_specs=[
        pl.BlockSpec((_RB, 64), lambda i: (i, 0)),
        pl.BlockSpec((_RB, 64), lambda i: (i, 0)),
        pl.BlockSpec((_RB, 16), lambda i: (i, 0)),
        pl.BlockSpec((_RB, 16), lambda i: (i, 0)),
        pl.BlockSpec((1, NCLS), lambda i: (0, 0)),
    ],
    out_specs=[
        pl.BlockSpec((_RB, NCLS), lambda i: (i, 0)),
        pl.BlockSpec((_RB, NCLS), lambda i: (i, 0)),
    ],
    out_shape=[
        jax.ShapeDtypeStruct((N, NCLS), F32),
        jax.ShapeDtypeStruct((N, NCLS), F32),
    ],
)


def kernel(x, edge_index, W1, b1, W2, b2, W3, b3):
    src2 = edge_index[0].reshape(ROWS, K)
    dst2 = edge_index[1].reshape(ROWS, K)
    ones8 = jnp.ones((K, 16), F32)
    zeros8 = jnp.zeros((WPT, 16), F32)
    zeros128 = jnp.zeros((K, 128), jnp.bfloat16)
    zeros64 = jnp.zeros((K, 64), jnp.bfloat16)
    W3p = jnp.zeros((D, 64), F32).at[:, :NCLS].set(W3)
    b1r = b1.reshape(1, D)
    b2r = b2.reshape(1, D)
    b3r = b3.reshape(1, NCLS)

    d0, d1 = _sc_deg(dst2, ones8, zeros8)
    y0, y1 = _tc_prescale(d0, d1, x)
    z0, z1 = _sc_apply_fs(y0, y1, src2, dst2, zeros128)
    emb1, y20, y21 = _tc_glue(z0, z1, d0, d1, W1, b1r)
    z20, z21 = _sc_apply_fs(y20, y21, src2, dst2, zeros128)
    emb2, y3 = _tc_glue3(z20, z21, d0, d1, W2, b2r, W3p)
    za, zb = _sc_apply_es(y3, src2, dst2, zeros64)
    out, emb3 = _tc_final(za, zb, d0, d1, b3r)
    return (out, emb1, emb2, emb3)
